# baseline (speedup 1.0000x reference)
import numpy as np
import jax
import jax.numpy as jnp
from jax import lax
from jax.experimental import pallas as pl
from jax.experimental.pallas import tpu as pltpu
from jax.experimental.pallas import tpu_sc as plsc

_N_LEVELS = 16
_MIN_RES = 16
_MAX_RES = 2048
_T_MAX = 2 ** 19
_GROWTH = np.exp((np.log(_MAX_RES) - np.log(_MIN_RES)) / (_N_LEVELS - 1))
_PRIME = 2654435761

_NW = 32
_P = 256
_CHUNK = 128
_L = 16
_SLOTS = 6
_DEPTH = 5
_N_RES = 8
_RES_ROWS = 49904


def _level_table():
    out = []
    off = 0
    for l in range(_N_LEVELS):
        scale = _MIN_RES * (_GROWTH ** l) - 1.0
        res = int(np.ceil(scale)) + 1
        dense = (res * res) <= _T_MAX
        size = res * res if dense else _T_MAX
        out.append((float(scale), res, dense, size, off))
        off += size
    return out, off


_LEVELS, _TOTAL_ROWS = _level_table()


def _encode_body(u_hbm, v_hbm, tab0_hbm, tab1_hbm, enc_hbm,
                 u_v, v_v, fx_v, fy_v, idx_a, idx_b, idx_c, idx_d, idx_e, idx_f,
                 rows0_a, rows0_b, rows0_c, rows0_d, rows0_e, rows0_f,
                 rows1_a, rows1_b, rows1_c, rows1_d, rows1_e, rows1_f,
                 enc_v, res0_v, res1_v,
                 sem0, sem1, sem2, sem3, sem4, sem5):
    wid = lax.axis_index("s") * 2 + lax.axis_index("c")
    npts = u_hbm.shape[0]
    per_w = npts // _NW
    nblk = per_w // _P
    iota = lax.iota(jnp.int32, _L)
    sems = (sem0, sem1, sem2, sem3, sem4, sem5)
    idxs = (idx_a, idx_b, idx_c, idx_d, idx_e, idx_f)
    rows0 = (rows0_a, rows0_b, rows0_c, rows0_d, rows0_e, rows0_f)
    rows1 = (rows1_a, rows1_b, rows1_c, rows1_d, rows1_e, rows1_f)
    nj = _P // _L

    pltpu.sync_copy(tab0_hbm.at[pl.ds(0, _RES_ROWS)], res0_v)
    pltpu.sync_copy(tab1_hbm.at[pl.ds(0, _RES_ROWS)], res1_v)

    def run_block(b, carry):
        base = wid * per_w + b * _P
        pltpu.sync_copy(u_hbm.at[pl.ds(base, _P)], u_v)
        pltpu.sync_copy(v_hbm.at[pl.ds(base, _P)], v_v)

        def pass_a(l, s):
            scale, res, dense, size, off = _LEVELS[l]

            def body(j, c):
                sl = pl.ds(j * _L, _L)
                u = u_v[sl]
                v = v_v[sl]
                px = u * scale + 0.5
                py = v * scale + 0.5
                ix = px.astype(jnp.int32)
                iy = py.astype(jnp.int32)
                fx_v[s, sl] = px - ix.astype(jnp.float32)
                fy_v[s, sl] = py - iy.astype(jnp.float32)
                if dense:
                    x0 = ix
                    x1 = jnp.minimum(ix + 1, res - 1)
                    y0 = iy * res + off
                    y1 = jnp.minimum(iy + 1, res - 1) * res + off
                    i00 = x0 + y0
                    i01 = x0 + y1
                    i10 = x1 + y0
                    i11 = x1 + y1
                else:
                    m = jnp.uint32(size - 1)
                    prime = jnp.uint32(_PRIME)
                    ux0 = ix.astype(jnp.uint32)
                    ux1 = ux0 + jnp.uint32(1)
                    t0 = iy.astype(jnp.uint32) * prime
                    t1 = t0 + prime
                    i00 = ((ux0 ^ t0) & m).astype(jnp.int32) + off
                    i01 = ((ux0 ^ t1) & m).astype(jnp.int32) + off
                    i10 = ((ux1 ^ t0) & m).astype(jnp.int32) + off
                    i11 = ((ux1 ^ t1) & m).astype(jnp.int32) + off
                idx_s = idxs[s]
                idx_s[pl.ds(j * _L, _L)] = i00
                idx_s[pl.ds(_P + j * _L, _L)] = i01
                idx_s[pl.ds(2 * _P + j * _L, _L)] = i10
                idx_s[pl.ds(3 * _P + j * _L, _L)] = i11
                return c

            lax.fori_loop(0, nj, body, 0, unroll=2)

        def fire(l, s):
            ds = []
            for k in range(4 * _P // _CHUNK):
                cs = pl.ds(k * _CHUNK, _CHUNK)
                ds.append(pltpu.async_copy(tab0_hbm.at[idxs[s].at[cs]], rows0[s].at[cs], sems[s]))
                ds.append(pltpu.async_copy(tab1_hbm.at[idxs[s].at[cs]], rows1[s].at[cs], sems[s]))
            return ds

        def pass_b(l, s):
            def body(j, c):
                sl = pl.ds(j * _L, _L)
                fx = fx_v[s, sl]
                fy = fy_v[s, sl]

                def g(c_off):
                    return (rows0[s][pl.ds(c_off + j * _L, _L)],
                            rows1[s][pl.ds(c_off + j * _L, _L)])

                g00a, g00b = g(0)
                g01a, g01b = g(_P)
                g10a, g10b = g(2 * _P)
                g11a, g11b = g(3 * _P)
                ea = g00a + fx * (g10a - g00a)
                eb = g01a + fx * (g11a - g01a)
                e0 = ea + fy * (eb - ea)
                ec = g00b + fx * (g10b - g00b)
                ed = g01b + fx * (g11b - g01b)
                e1 = ec + fy * (ed - ec)
                enc_v[2 * l, sl] = e0
                enc_v[2 * l + 1, sl] = e1
                return c

            lax.fori_loop(0, nj, body, 0, unroll=2)

        def fused_res(l):
            scale, res, dense, size, off = _LEVELS[l]

            def body(j, c):
                sl = pl.ds(j * _L, _L)
                u = u_v[sl]
                v = v_v[sl]
                px = u * scale + 0.5
                py = v * scale + 0.5
                ix = px.astype(jnp.int32)
                iy = py.astype(jnp.int32)
                fx = px - ix.astype(jnp.float32)
                fy = py - iy.astype(jnp.float32)
                x0 = ix
                x1 = jnp.minimum(ix + 1, res - 1)
                y0 = iy * res + off
                y1 = jnp.minimum(iy + 1, res - 1) * res + off
                i00 = x0 + y0
                i01 = x0 + y1
                i10 = x1 + y0
                i11 = x1 + y1
                g00a = plsc.load_gather(res0_v, [i00])
                g00b = plsc.load_gather(res1_v, [i00])
                g01a = plsc.load_gather(res0_v, [i01])
                g01b = plsc.load_gather(res1_v, [i01])
                g10a = plsc.load_gather(res0_v, [i10])
                g10b = plsc.load_gather(res1_v, [i10])
                g11a = plsc.load_gather(res0_v, [i11])
                g11b = plsc.load_gather(res1_v, [i11])
                ea = g00a + fx * (g10a - g00a)
                eb = g01a + fx * (g11a - g01a)
                e0 = ea + fy * (eb - ea)
                ec = g00b + fx * (g10b - g00b)
                ed = g01b + fx * (g11b - g01b)
                e1 = ec + fy * (ed - ec)
                enc_v[2 * l, sl] = e0
                enc_v[2 * l + 1, sl] = e1
                return c

            lax.fori_loop(0, nj, body, 0, unroll=2)

        lvls = list(range(_N_RES, _N_LEVELS))
        descs = [None] * _SLOTS
        for d in range(_DEPTH):
            pass_a(lvls[d], d % _SLOTS)
            descs[d % _SLOTS] = fire(lvls[d], d % _SLOTS)
        for l in range(_N_RES):
            fused_res(l)
        for i, l in enumerate(lvls):
            s = i % _SLOTS
            if i + _DEPTH < len(lvls):
                s2 = (i + _DEPTH) % _SLOTS
                pass_a(lvls[i + _DEPTH], s2)
                descs[s2] = fire(lvls[i + _DEPTH], s2)
            for dsc in descs[s]:
                dsc.wait()
            pass_b(l, s)
        pltpu.sync_copy(enc_v, enc_hbm.at[:, pl.ds(base, _P)])
        return carry

    lax.fori_loop(0, nblk, run_block, 0)


def _encode(u, v, tab0, tab1, interpret=False):
    n = u.shape[0]
    mesh = plsc.VectorSubcoreMesh(core_axis_name="c", subcore_axis_name="s",
                                  num_cores=2, num_subcores=16)
    scratch = [
        pltpu.VMEM((_P,), jnp.float32),
        pltpu.VMEM((_P,), jnp.float32),
        pltpu.VMEM((_SLOTS, _P), jnp.float32),
        pltpu.VMEM((_SLOTS, _P), jnp.float32),
        pltpu.VMEM((4 * _P,), jnp.int32),
        pltpu.VMEM((4 * _P,), jnp.int32),
        pltpu.VMEM((4 * _P,), jnp.int32),
        pltpu.VMEM((4 * _P,), jnp.int32),
        pltpu.VMEM((4 * _P,), jnp.int32),
        pltpu.VMEM((4 * _P,), jnp.int32),
        pltpu.VMEM((4 * _P,), jnp.float32),
        pltpu.VMEM((4 * _P,), jnp.float32),
        pltpu.VMEM((4 * _P,), jnp.float32),
        pltpu.VMEM((4 * _P,), jnp.float32),
        pltpu.VMEM((4 * _P,), jnp.float32),
        pltpu.VMEM((4 * _P,), jnp.float32),
        pltpu.VMEM((4 * _P,), jnp.float32),
        pltpu.VMEM((4 * _P,), jnp.float32),
        pltpu.VMEM((4 * _P,), jnp.float32),
        pltpu.VMEM((4 * _P,), jnp.float32),
        pltpu.VMEM((4 * _P,), jnp.float32),
        pltpu.VMEM((4 * _P,), jnp.float32),
        pltpu.VMEM((2 * _N_LEVELS, _P), jnp.float32),
        pltpu.VMEM((_RES_ROWS,), jnp.float32),
        pltpu.VMEM((_RES_ROWS,), jnp.float32),
        pltpu.SemaphoreType.DMA,
        pltpu.SemaphoreType.DMA,
        pltpu.SemaphoreType.DMA,
        pltpu.SemaphoreType.DMA,
        pltpu.SemaphoreType.DMA,
        pltpu.SemaphoreType.DMA,
    ]
    f = pl.kernel(
        _encode_body,
        out_type=jax.ShapeDtypeStruct((2 * _N_LEVELS, n), jnp.float32),
        mesh=mesh,
        scratch_types=scratch,
        compiler_params=pltpu.CompilerParams(needs_layout_passes=False),
        interpret=interpret,
    )
    return f(u, v, tab0, tab1)


def _mlp(enc_t, W0, W1, W2p, interpret=False):
    n = enc_t.shape[1]
    bn = 2048

    def body(x_ref, w0_ref, w1_ref, w2_ref, o_ref):
        x = x_ref[...]
        h0 = lax.dot_general(x, w0_ref[...], (((0,), (0,)), ((), ())),
                             preferred_element_type=jnp.float32)
        h0 = jnp.maximum(h0, 0.0)
        h1 = jnp.maximum(jnp.dot(h0, w1_ref[...], preferred_element_type=jnp.float32), 0.0)
        z = jnp.dot(h1, w2_ref[...], preferred_element_type=jnp.float32)
        o_ref[...] = 1.0 / (1.0 + jnp.exp(-z))

    return pl.pallas_call(
        body,
        grid=(n // bn,),
        in_specs=[
            pl.BlockSpec((2 * _N_LEVELS, bn), lambda i: (0, i)),
            pl.BlockSpec((2 * _N_LEVELS, 64), lambda i: (0, 0)),
            pl.BlockSpec((64, 64), lambda i: (0, 0)),
            pl.BlockSpec((64, 8), lambda i: (0, 0)),
        ],
        out_specs=pl.BlockSpec((bn, 8), lambda i: (i, 0)),
        out_shape=jax.ShapeDtypeStruct((n, 8), jnp.float32),
        interpret=interpret,
    )(enc_t, W0, W1, W2p)


def kernel(uv, W0, W1, W2, table_0, table_1, table_2, table_3, table_4,
           table_5, table_6, table_7, table_8, table_9, table_10, table_11,
           table_12, table_13, table_14, table_15):
    tables = [table_0, table_1, table_2, table_3, table_4, table_5, table_6,
              table_7, table_8, table_9, table_10, table_11, table_12,
              table_13, table_14, table_15]
    tab = jnp.concatenate(tables, axis=0)
    tab0 = tab[:, 0] + 0.0
    tab1 = tab[:, 1] + 0.0
    u = uv[:, 0] + 0.0
    v = uv[:, 1] + 0.0
    enc_t = _encode(u, v, tab0, tab1)
    w2p = jnp.pad(W2, ((0, 0), (0, 8 - W2.shape[1])))
    out = _mlp(enc_t, W0, W1, w2p)
    return out[:, : W2.shape[1]]


# speedup vs baseline: 1.0450x; 1.0450x over previous
import numpy as np
import jax
import jax.numpy as jnp
from jax import lax
from jax.experimental import pallas as pl
from jax.experimental.pallas import tpu as pltpu
from jax.experimental.pallas import tpu_sc as plsc

_N_LEVELS = 16
_MIN_RES = 16
_MAX_RES = 2048
_T_MAX = 2 ** 19
_GROWTH = np.exp((np.log(_MAX_RES) - np.log(_MIN_RES)) / (_N_LEVELS - 1))
_PRIME = 2654435761

_NW = 32
_P = 256
_CHUNK = 128
_L = 16
_SLOTS = 6
_DEPTH = 5
_N_RES = 8
_RES_ROWS = 49904


def _level_table():
    out = []
    off = 0
    for l in range(_N_LEVELS):
        scale = _MIN_RES * (_GROWTH ** l) - 1.0
        res = int(np.ceil(scale)) + 1
        dense = (res * res) <= _T_MAX
        size = res * res if dense else _T_MAX
        out.append((float(scale), res, dense, size, off))
        off += size
    return out, off


_LEVELS, _TOTAL_ROWS = _level_table()


def _encode_body(u_hbm, v_hbm, tab0_hbm, tab1_hbm, enc_hbm,
                 u_v, v_v, fx_v, fy_v, idx_a, idx_b, idx_c, idx_d, idx_e, idx_f,
                 rows0_a, rows0_b, rows0_c, rows0_d, rows0_e, rows0_f,
                 rows1_a, rows1_b, rows1_c, rows1_d, rows1_e, rows1_f,
                 enc_v, res0_v, res1_v,
                 sem0, sem1, sem2, sem3, sem4, sem5):
    wid = lax.axis_index("s") * 2 + lax.axis_index("c")
    npts = u_hbm.shape[0]
    per_w = npts // _NW
    nblk = per_w // _P
    iota = lax.iota(jnp.int32, _L)
    sems = (sem0, sem1, sem2, sem3, sem4, sem5)
    idxs = (idx_a, idx_b, idx_c, idx_d, idx_e, idx_f)
    rows0 = (rows0_a, rows0_b, rows0_c, rows0_d, rows0_e, rows0_f)
    rows1 = (rows1_a, rows1_b, rows1_c, rows1_d, rows1_e, rows1_f)
    nj = _P // _L

    pltpu.sync_copy(tab0_hbm.at[pl.ds(0, _RES_ROWS)], res0_v)
    pltpu.sync_copy(tab1_hbm.at[pl.ds(0, _RES_ROWS)], res1_v)

    def run_block(b, carry):
        base = wid * per_w + b * _P
        pltpu.sync_copy(u_hbm.at[pl.ds(base, _P)], u_v)
        pltpu.sync_copy(v_hbm.at[pl.ds(base, _P)], v_v)

        def pass_a(l, s):
            scale, res, dense, size, off = _LEVELS[l]

            def body(j, c):
                sl = pl.ds(j * _L, _L)
                u = u_v[sl]
                v = v_v[sl]
                px = u * scale + 0.5
                py = v * scale + 0.5
                ix = px.astype(jnp.int32)
                iy = py.astype(jnp.int32)
                fx_v[s, sl] = px - ix.astype(jnp.float32)
                fy_v[s, sl] = py - iy.astype(jnp.float32)
                if dense:
                    x0 = ix
                    x1 = jnp.minimum(ix + 1, res - 1)
                    y0 = iy * res + off
                    y1 = jnp.minimum(iy + 1, res - 1) * res + off
                    i00 = x0 + y0
                    i01 = x0 + y1
                    i10 = x1 + y0
                    i11 = x1 + y1
                else:
                    m = jnp.uint32(size - 1)
                    prime = jnp.uint32(_PRIME)
                    ux0 = ix.astype(jnp.uint32)
                    ux1 = ux0 + jnp.uint32(1)
                    t0 = iy.astype(jnp.uint32) * prime
                    t1 = t0 + prime
                    i00 = ((ux0 ^ t0) & m).astype(jnp.int32) + off
                    i01 = ((ux0 ^ t1) & m).astype(jnp.int32) + off
                    i10 = ((ux1 ^ t0) & m).astype(jnp.int32) + off
                    i11 = ((ux1 ^ t1) & m).astype(jnp.int32) + off
                idx_s = idxs[s]
                idx_s[pl.ds(j * _L, _L)] = i00
                idx_s[pl.ds(_P + j * _L, _L)] = i01
                idx_s[pl.ds(2 * _P + j * _L, _L)] = i10
                idx_s[pl.ds(3 * _P + j * _L, _L)] = i11
                return c

            lax.fori_loop(0, nj, body, 0, unroll=2)

        def fire(l, s):
            ds = []
            for k in range(4 * _P // _CHUNK):
                cs = pl.ds(k * _CHUNK, _CHUNK)
                ds.append(pltpu.async_copy(tab0_hbm.at[idxs[s].at[cs]], rows0[s].at[cs], sems[s]))
                ds.append(pltpu.async_copy(tab1_hbm.at[idxs[s].at[cs]], rows1[s].at[cs], sems[s]))
            return ds

        def pass_b(l, s):
            def body(j, c):
                sl = pl.ds(j * _L, _L)
                fx = fx_v[s, sl]
                fy = fy_v[s, sl]

                def g(c_off):
                    return (rows0[s][pl.ds(c_off + j * _L, _L)],
                            rows1[s][pl.ds(c_off + j * _L, _L)])

                g00a, g00b = g(0)
                g01a, g01b = g(_P)
                g10a, g10b = g(2 * _P)
                g11a, g11b = g(3 * _P)
                ea = g00a + fx * (g10a - g00a)
                eb = g01a + fx * (g11a - g01a)
                e0 = ea + fy * (eb - ea)
                ec = g00b + fx * (g10b - g00b)
                ed = g01b + fx * (g11b - g01b)
                e1 = ec + fy * (ed - ec)
                enc_v[2 * l, sl] = e0
                enc_v[2 * l + 1, sl] = e1
                return c

            lax.fori_loop(0, nj, body, 0, unroll=2)

        def fused_res(l):
            scale, res, dense, size, off = _LEVELS[l]

            def body(j, c):
                sl = pl.ds(j * _L, _L)
                u = u_v[sl]
                v = v_v[sl]
                px = u * scale + 0.5
                py = v * scale + 0.5
                ix = px.astype(jnp.int32)
                iy = py.astype(jnp.int32)
                fx = px - ix.astype(jnp.float32)
                fy = py - iy.astype(jnp.float32)
                x0 = ix
                x1 = jnp.minimum(ix + 1, res - 1)
                y0 = iy * res + off
                y1 = jnp.minimum(iy + 1, res - 1) * res + off
                i00 = x0 + y0
                i01 = x0 + y1
                i10 = x1 + y0
                i11 = x1 + y1
                g00a = plsc.load_gather(res0_v, [i00])
                g00b = plsc.load_gather(res1_v, [i00])
                g01a = plsc.load_gather(res0_v, [i01])
                g01b = plsc.load_gather(res1_v, [i01])
                g10a = plsc.load_gather(res0_v, [i10])
                g10b = plsc.load_gather(res1_v, [i10])
                g11a = plsc.load_gather(res0_v, [i11])
                g11b = plsc.load_gather(res1_v, [i11])
                ea = g00a + fx * (g10a - g00a)
                eb = g01a + fx * (g11a - g01a)
                e0 = ea + fy * (eb - ea)
                ec = g00b + fx * (g10b - g00b)
                ed = g01b + fx * (g11b - g01b)
                e1 = ec + fy * (ed - ec)
                enc_v[2 * l, sl] = e0
                enc_v[2 * l + 1, sl] = e1
                return c

            lax.fori_loop(0, nj, body, 0, unroll=2)

        lvls = list(range(_N_RES, _N_LEVELS))
        descs = [None] * _SLOTS
        for d in range(_DEPTH):
            pass_a(lvls[d], d % _SLOTS)
            descs[d % _SLOTS] = fire(lvls[d], d % _SLOTS)
        for l in range(_N_RES):
            fused_res(l)
        for i, l in enumerate(lvls):
            s = i % _SLOTS
            if i + _DEPTH < len(lvls):
                s2 = (i + _DEPTH) % _SLOTS
                pass_a(lvls[i + _DEPTH], s2)
                descs[s2] = fire(lvls[i + _DEPTH], s2)
            for dsc in descs[s]:
                dsc.wait()
            pass_b(l, s)
        pltpu.sync_copy(enc_v, enc_hbm.at[:, pl.ds(base, _P)])
        return carry

    lax.fori_loop(0, nblk, run_block, 0)


def _encode(u, v, tab0, tab1, interpret=False):
    n = u.shape[0]
    mesh = plsc.VectorSubcoreMesh(core_axis_name="c", subcore_axis_name="s",
                                  num_cores=2, num_subcores=16)
    scratch = [
        pltpu.VMEM((_P,), jnp.float32),
        pltpu.VMEM((_P,), jnp.float32),
        pltpu.VMEM((_SLOTS, _P), jnp.float32),
        pltpu.VMEM((_SLOTS, _P), jnp.float32),
        pltpu.VMEM((4 * _P,), jnp.int32),
        pltpu.VMEM((4 * _P,), jnp.int32),
        pltpu.VMEM((4 * _P,), jnp.int32),
        pltpu.VMEM((4 * _P,), jnp.int32),
        pltpu.VMEM((4 * _P,), jnp.int32),
        pltpu.VMEM((4 * _P,), jnp.int32),
        pltpu.VMEM((4 * _P,), jnp.float32),
        pltpu.VMEM((4 * _P,), jnp.float32),
        pltpu.VMEM((4 * _P,), jnp.float32),
        pltpu.VMEM((4 * _P,), jnp.float32),
        pltpu.VMEM((4 * _P,), jnp.float32),
        pltpu.VMEM((4 * _P,), jnp.float32),
        pltpu.VMEM((4 * _P,), jnp.float32),
        pltpu.VMEM((4 * _P,), jnp.float32),
        pltpu.VMEM((4 * _P,), jnp.float32),
        pltpu.VMEM((4 * _P,), jnp.float32),
        pltpu.VMEM((4 * _P,), jnp.float32),
        pltpu.VMEM((4 * _P,), jnp.float32),
        pltpu.VMEM((2 * _N_LEVELS, _P), jnp.float32),
        pltpu.VMEM((_RES_ROWS,), jnp.float32),
        pltpu.VMEM((_RES_ROWS,), jnp.float32),
        pltpu.SemaphoreType.DMA,
        pltpu.SemaphoreType.DMA,
        pltpu.SemaphoreType.DMA,
        pltpu.SemaphoreType.DMA,
        pltpu.SemaphoreType.DMA,
        pltpu.SemaphoreType.DMA,
    ]
    f = pl.kernel(
        _encode_body,
        out_type=jax.ShapeDtypeStruct((2 * _N_LEVELS, n), jnp.float32),
        mesh=mesh,
        scratch_types=scratch,
        compiler_params=pltpu.CompilerParams(needs_layout_passes=False),
        interpret=interpret,
    )
    return f(u, v, tab0, tab1)


def _mlp(enc_t, W0, W1, W2p, interpret=False):
    n = enc_t.shape[1]
    bn = 2048

    def body(x_ref, w0_ref, w1_ref, w2_ref, o_ref):
        x = x_ref[...]
        h0 = lax.dot_general(x, w0_ref[...], (((0,), (0,)), ((), ())),
                             preferred_element_type=jnp.float32)
        h0 = jnp.maximum(h0, 0.0)
        h1 = jnp.maximum(jnp.dot(h0, w1_ref[...], preferred_element_type=jnp.float32), 0.0)
        z = jnp.dot(h1, w2_ref[...], preferred_element_type=jnp.float32)
        o_ref[...] = 1.0 / (1.0 + jnp.exp(-z))

    return pl.pallas_call(
        body,
        grid=(n // bn,),
        in_specs=[
            pl.BlockSpec((2 * _N_LEVELS, bn), lambda i: (0, i)),
            pl.BlockSpec((2 * _N_LEVELS, 64), lambda i: (0, 0)),
            pl.BlockSpec((64, 64), lambda i: (0, 0)),
            pl.BlockSpec((64, 8), lambda i: (0, 0)),
        ],
        out_specs=pl.BlockSpec((bn, 8), lambda i: (i, 0)),
        out_shape=jax.ShapeDtypeStruct((n, 8), jnp.float32),
        interpret=interpret,
    )(enc_t, W0, W1, W2p)


def kernel(uv, W0, W1, W2, table_0, table_1, table_2, table_3, table_4,
           table_5, table_6, table_7, table_8, table_9, table_10, table_11,
           table_12, table_13, table_14, table_15):
    tables = [table_0, table_1, table_2, table_3, table_4, table_5, table_6,
              table_7, table_8, table_9, table_10, table_11, table_12,
              table_13, table_14, table_15]
    tab = jnp.concatenate(tables, axis=0)
    tab0 = tab[:, 0] + 0.0
    tab1 = tab[:, 1] + 0.0
    u = uv[:, 0] + 0.0
    v = uv[:, 1] + 0.0
    w2p = jnp.pad(W2, ((0, 0), (0, 8 - W2.shape[1])))
    h = u.shape[0] // 2
    enc0 = _encode(u[:h], v[:h], tab0, tab1)
    out0 = _mlp(enc0, W0, W1, w2p)
    enc1 = _encode(u[h:], v[h:], tab0, tab1)
    out1 = _mlp(enc1, W0, W1, w2p)
    out = jnp.concatenate([out0, out1], axis=0)
    return out[:, : W2.shape[1]]


# speedup vs baseline: 1.0476x; 1.0024x over previous
import numpy as np
import jax
import jax.numpy as jnp
from jax import lax
from jax.experimental import pallas as pl
from jax.experimental.pallas import tpu as pltpu
from jax.experimental.pallas import tpu_sc as plsc

_N_LEVELS = 16
_MIN_RES = 16
_MAX_RES = 2048
_T_MAX = 2 ** 19
_GROWTH = np.exp((np.log(_MAX_RES) - np.log(_MIN_RES)) / (_N_LEVELS - 1))
_PRIME = 2654435761

_NW = 32
_P = 256
_CHUNK = 128
_L = 16
_SLOTS = 6
_DEPTH = 5
_N_RES = 8
_RES_ROWS = 49904


def _level_table():
    out = []
    off = 0
    for l in range(_N_LEVELS):
        scale = _MIN_RES * (_GROWTH ** l) - 1.0
        res = int(np.ceil(scale)) + 1
        dense = (res * res) <= _T_MAX
        size = res * res if dense else _T_MAX
        out.append((float(scale), res, dense, size, off))
        off += size
    return out, off


_LEVELS, _TOTAL_ROWS = _level_table()


def _encode_body(u_hbm, v_hbm, tab0_hbm, tab1_hbm, enc_hbm,
                 u_v, v_v, fx_v, fy_v, idx_a, idx_b, idx_c, idx_d, idx_e, idx_f,
                 rows0_a, rows0_b, rows0_c, rows0_d, rows0_e, rows0_f,
                 rows1_a, rows1_b, rows1_c, rows1_d, rows1_e, rows1_f,
                 enc_v, res0_v, res1_v,
                 sem0, sem1, sem2, sem3, sem4, sem5):
    wid = lax.axis_index("s") * 2 + lax.axis_index("c")
    npts = u_hbm.shape[0]
    per_w = npts // _NW
    nblk = per_w // _P
    iota = lax.iota(jnp.int32, _L)
    sems = (sem0, sem1, sem2, sem3, sem4, sem5)
    idxs = (idx_a, idx_b, idx_c, idx_d, idx_e, idx_f)
    rows0 = (rows0_a, rows0_b, rows0_c, rows0_d, rows0_e, rows0_f)
    rows1 = (rows1_a, rows1_b, rows1_c, rows1_d, rows1_e, rows1_f)
    nj = _P // _L

    pltpu.sync_copy(tab0_hbm.at[pl.ds(0, _RES_ROWS)], res0_v)
    pltpu.sync_copy(tab1_hbm.at[pl.ds(0, _RES_ROWS)], res1_v)

    def run_block(b, carry):
        base = wid * per_w + b * _P
        pltpu.sync_copy(u_hbm.at[pl.ds(base, _P)], u_v)
        pltpu.sync_copy(v_hbm.at[pl.ds(base, _P)], v_v)

        def pass_a(l, s):
            scale, res, dense, size, off = _LEVELS[l]

            def body(j, c):
                sl = pl.ds(j * _L, _L)
                u = u_v[sl]
                v = v_v[sl]
                px = u * scale + 0.5
                py = v * scale + 0.5
                ix = px.astype(jnp.int32)
                iy = py.astype(jnp.int32)
                fx_v[s, sl] = px - ix.astype(jnp.float32)
                fy_v[s, sl] = py - iy.astype(jnp.float32)
                if dense:
                    x0 = ix
                    x1 = jnp.minimum(ix + 1, res - 1)
                    y0 = iy * res + off
                    y1 = jnp.minimum(iy + 1, res - 1) * res + off
                    i00 = x0 + y0
                    i01 = x0 + y1
                    i10 = x1 + y0
                    i11 = x1 + y1
                else:
                    m = jnp.uint32(size - 1)
                    prime = jnp.uint32(_PRIME)
                    ux0 = ix.astype(jnp.uint32)
                    ux1 = ux0 + jnp.uint32(1)
                    t0 = iy.astype(jnp.uint32) * prime
                    t1 = t0 + prime
                    i00 = ((ux0 ^ t0) & m).astype(jnp.int32) + off
                    i01 = ((ux0 ^ t1) & m).astype(jnp.int32) + off
                    i10 = ((ux1 ^ t0) & m).astype(jnp.int32) + off
                    i11 = ((ux1 ^ t1) & m).astype(jnp.int32) + off
                idx_s = idxs[s]
                idx_s[pl.ds(j * _L, _L)] = i00
                idx_s[pl.ds(_P + j * _L, _L)] = i01
                idx_s[pl.ds(2 * _P + j * _L, _L)] = i10
                idx_s[pl.ds(3 * _P + j * _L, _L)] = i11
                return c

            lax.fori_loop(0, nj, body, 0, unroll=2)

        def fire(l, s):
            ds = []
            for k in range(4 * _P // _CHUNK):
                cs = pl.ds(k * _CHUNK, _CHUNK)
                ds.append(pltpu.async_copy(tab0_hbm.at[idxs[s].at[cs]], rows0[s].at[cs], sems[s]))
                ds.append(pltpu.async_copy(tab1_hbm.at[idxs[s].at[cs]], rows1[s].at[cs], sems[s]))
            return ds

        def pass_b(l, s):
            def body(j, c):
                sl = pl.ds(j * _L, _L)
                fx = fx_v[s, sl]
                fy = fy_v[s, sl]

                def g(c_off):
                    return (rows0[s][pl.ds(c_off + j * _L, _L)],
                            rows1[s][pl.ds(c_off + j * _L, _L)])

                g00a, g00b = g(0)
                g01a, g01b = g(_P)
                g10a, g10b = g(2 * _P)
                g11a, g11b = g(3 * _P)
                ea = g00a + fx * (g10a - g00a)
                eb = g01a + fx * (g11a - g01a)
                e0 = ea + fy * (eb - ea)
                ec = g00b + fx * (g10b - g00b)
                ed = g01b + fx * (g11b - g01b)
                e1 = ec + fy * (ed - ec)
                enc_v[2 * l, sl] = e0
                enc_v[2 * l + 1, sl] = e1
                return c

            lax.fori_loop(0, nj, body, 0, unroll=2)

        def fused_res(l):
            scale, res, dense, size, off = _LEVELS[l]

            def body(j, c):
                sl = pl.ds(j * _L, _L)
                u = u_v[sl]
                v = v_v[sl]
                px = u * scale + 0.5
                py = v * scale + 0.5
                ix = px.astype(jnp.int32)
                iy = py.astype(jnp.int32)
                fx = px - ix.astype(jnp.float32)
                fy = py - iy.astype(jnp.float32)
                x0 = ix
                x1 = jnp.minimum(ix + 1, res - 1)
                y0 = iy * res + off
                y1 = jnp.minimum(iy + 1, res - 1) * res + off
                i00 = x0 + y0
                i01 = x0 + y1
                i10 = x1 + y0
                i11 = x1 + y1
                g00a = plsc.load_gather(res0_v, [i00])
                g00b = plsc.load_gather(res1_v, [i00])
                g01a = plsc.load_gather(res0_v, [i01])
                g01b = plsc.load_gather(res1_v, [i01])
                g10a = plsc.load_gather(res0_v, [i10])
                g10b = plsc.load_gather(res1_v, [i10])
                g11a = plsc.load_gather(res0_v, [i11])
                g11b = plsc.load_gather(res1_v, [i11])
                ea = g00a + fx * (g10a - g00a)
                eb = g01a + fx * (g11a - g01a)
                e0 = ea + fy * (eb - ea)
                ec = g00b + fx * (g10b - g00b)
                ed = g01b + fx * (g11b - g01b)
                e1 = ec + fy * (ed - ec)
                enc_v[2 * l, sl] = e0
                enc_v[2 * l + 1, sl] = e1
                return c

            lax.fori_loop(0, nj, body, 0, unroll=2)

        lvls = list(range(_N_RES, _N_LEVELS))
        descs = [None] * _SLOTS
        for d in range(_DEPTH):
            pass_a(lvls[d], d % _SLOTS)
            descs[d % _SLOTS] = fire(lvls[d], d % _SLOTS)
        for l in range(_N_RES):
            fused_res(l)
        for i, l in enumerate(lvls):
            s = i % _SLOTS
            if i + _DEPTH < len(lvls):
                s2 = (i + _DEPTH) % _SLOTS
                pass_a(lvls[i + _DEPTH], s2)
                descs[s2] = fire(lvls[i + _DEPTH], s2)
            for dsc in descs[s]:
                dsc.wait()
            pass_b(l, s)
        pltpu.sync_copy(enc_v, enc_hbm.at[:, pl.ds(base, _P)])
        return carry

    lax.fori_loop(0, nblk, run_block, 0)


def _encode(u, v, tab0, tab1, interpret=False):
    n = u.shape[0]
    mesh = plsc.VectorSubcoreMesh(core_axis_name="c", subcore_axis_name="s",
                                  num_cores=2, num_subcores=16)
    scratch = [
        pltpu.VMEM((_P,), jnp.float32),
        pltpu.VMEM((_P,), jnp.float32),
        pltpu.VMEM((_SLOTS, _P), jnp.float32),
        pltpu.VMEM((_SLOTS, _P), jnp.float32),
        pltpu.VMEM((4 * _P,), jnp.int32),
        pltpu.VMEM((4 * _P,), jnp.int32),
        pltpu.VMEM((4 * _P,), jnp.int32),
        pltpu.VMEM((4 * _P,), jnp.int32),
        pltpu.VMEM((4 * _P,), jnp.int32),
        pltpu.VMEM((4 * _P,), jnp.int32),
        pltpu.VMEM((4 * _P,), jnp.float32),
        pltpu.VMEM((4 * _P,), jnp.float32),
        pltpu.VMEM((4 * _P,), jnp.float32),
        pltpu.VMEM((4 * _P,), jnp.float32),
        pltpu.VMEM((4 * _P,), jnp.float32),
        pltpu.VMEM((4 * _P,), jnp.float32),
        pltpu.VMEM((4 * _P,), jnp.float32),
        pltpu.VMEM((4 * _P,), jnp.float32),
        pltpu.VMEM((4 * _P,), jnp.float32),
        pltpu.VMEM((4 * _P,), jnp.float32),
        pltpu.VMEM((4 * _P,), jnp.float32),
        pltpu.VMEM((4 * _P,), jnp.float32),
        pltpu.VMEM((2 * _N_LEVELS, _P), jnp.float32),
        pltpu.VMEM((_RES_ROWS,), jnp.float32),
        pltpu.VMEM((_RES_ROWS,), jnp.float32),
        pltpu.SemaphoreType.DMA,
        pltpu.SemaphoreType.DMA,
        pltpu.SemaphoreType.DMA,
        pltpu.SemaphoreType.DMA,
        pltpu.SemaphoreType.DMA,
        pltpu.SemaphoreType.DMA,
    ]
    f = pl.kernel(
        _encode_body,
        out_type=jax.ShapeDtypeStruct((2 * _N_LEVELS, n), jnp.float32),
        mesh=mesh,
        scratch_types=scratch,
        compiler_params=pltpu.CompilerParams(needs_layout_passes=False),
        interpret=interpret,
    )
    return f(u, v, tab0, tab1)


def _mlp(enc_t, W0, W1, W2p, interpret=False):
    n = enc_t.shape[1]
    bn = 2048

    def body(x_ref, w0_ref, w1_ref, w2_ref, o_ref):
        x = x_ref[...]
        h0 = lax.dot_general(x, w0_ref[...], (((0,), (0,)), ((), ())),
                             preferred_element_type=jnp.float32)
        h0 = jnp.maximum(h0, 0.0)
        h1 = jnp.maximum(jnp.dot(h0, w1_ref[...], preferred_element_type=jnp.float32), 0.0)
        z = jnp.dot(h1, w2_ref[...], preferred_element_type=jnp.float32)
        o_ref[...] = 1.0 / (1.0 + jnp.exp(-z))

    return pl.pallas_call(
        body,
        grid=(n // bn,),
        in_specs=[
            pl.BlockSpec((2 * _N_LEVELS, bn), lambda i: (0, i)),
            pl.BlockSpec((2 * _N_LEVELS, 64), lambda i: (0, 0)),
            pl.BlockSpec((64, 64), lambda i: (0, 0)),
            pl.BlockSpec((64, 8), lambda i: (0, 0)),
        ],
        out_specs=pl.BlockSpec((bn, 8), lambda i: (i, 0)),
        out_shape=jax.ShapeDtypeStruct((n, 8), jnp.float32),
        interpret=interpret,
    )(enc_t, W0, W1, W2p)


def kernel(uv, W0, W1, W2, table_0, table_1, table_2, table_3, table_4,
           table_5, table_6, table_7, table_8, table_9, table_10, table_11,
           table_12, table_13, table_14, table_15):
    tables = [table_0, table_1, table_2, table_3, table_4, table_5, table_6,
              table_7, table_8, table_9, table_10, table_11, table_12,
              table_13, table_14, table_15]
    tab = jnp.concatenate(tables, axis=0)
    tab0 = tab[:, 0] + 0.0
    tab1 = tab[:, 1] + 0.0
    u = uv[:, 0] + 0.0
    v = uv[:, 1] + 0.0
    w2p = jnp.pad(W2, ((0, 0), (0, 8 - W2.shape[1])))
    nsplit = 4
    h = u.shape[0] // nsplit
    outs = []
    for k in range(nsplit):
        enc_k = _encode(u[k * h:(k + 1) * h], v[k * h:(k + 1) * h], tab0, tab1)
        outs.append(_mlp(enc_k, W0, W1, w2p))
    out = jnp.concatenate(outs, axis=0)
    return out[:, : W2.shape[1]]
